# baseline (speedup 1.0000x reference)
import jax
import jax.numpy as jnp
from jax.experimental import pallas as pl

_B1 = 64
_B3 = 32


def _rowsum_kernel(x0_ref, x1_ref, x2_ref, x3_ref, out_ref):
    b = x0_ref.shape[0]
    for k, xr in enumerate((x0_ref, x1_ref, x2_ref, x3_ref)):
        out_ref[k * b:(k + 1) * b, :] = jnp.sum(xr[...], axis=1, keepdims=True)


def _median_kernel(c_ref, out_ref):
    c = c_ref[...]
    bits = jax.lax.bitcast_convert_type(c, jnp.int32)
    pos = bits > 0
    n_pos = jnp.sum(pos.astype(jnp.int32))
    target = (n_pos - 1) // 2 + 1

    def body(i, lo):
        cand = lo + (jnp.int32(1) << (30 - i))
        g = jnp.sum((pos & (bits < cand)).astype(jnp.int32))
        return jnp.where(g >= target, lo, cand)

    ans = jax.lax.fori_loop(0, 31, body, jnp.int32(0))
    after = jax.lax.bitcast_convert_type(ans, jnp.float32)
    after = jnp.where(n_pos == 0, jnp.inf, after)
    out_ref[...] = jnp.full(out_ref.shape, after, dtype=out_ref.dtype)


def _finalize_kernel(x0_ref, x1_ref, x2_ref, x3_ref, c_ref, after_ref, out_ref):
    b = x0_ref.shape[0]
    g = x0_ref.shape[1]
    c = c_ref[...]
    c_adj = c + (c == 0.0).astype(c.dtype)
    scale = after_ref[0, 0] / c_adj
    t = jnp.log10(c_adj)
    for k, xr in enumerate((x0_ref, x1_ref, x2_ref, x3_ref)):
        sl = slice(k * b, (k + 1) * b)
        out_ref[sl, :g] = jnp.log1p(xr[...] * scale[sl, :])
    out_ref[:, g:] = jnp.broadcast_to(t, (t.shape[0], 2))


def kernel(X):
    X = jnp.squeeze(X)
    n, g = X.shape

    counts = pl.pallas_call(
        _rowsum_kernel,
        grid=(n // (4 * _B1),),
        in_specs=[
            pl.BlockSpec((_B1, g), lambda i, k=k: (4 * i + k, 0))
            for k in range(4)
        ],
        out_specs=pl.BlockSpec((4 * _B1, 1), lambda i: (i, 0)),
        out_shape=jax.ShapeDtypeStruct((n, 1), X.dtype),
    )(X, X, X, X)

    after = pl.pallas_call(
        _median_kernel,
        out_shape=jax.ShapeDtypeStruct((1, 1), X.dtype),
    )(counts.reshape(n // 128, 128))

    out = pl.pallas_call(
        _finalize_kernel,
        grid=(n // (4 * _B3),),
        in_specs=[
            pl.BlockSpec((_B3, g), lambda i, k=k: (4 * i + k, 0))
            for k in range(4)
        ] + [
            pl.BlockSpec((4 * _B3, 1), lambda i: (i, 0)),
            pl.BlockSpec((1, 1), lambda i: (0, 0)),
        ],
        out_specs=pl.BlockSpec((4 * _B3, g + 2), lambda i: (i, 0)),
        out_shape=jax.ShapeDtypeStruct((n, g + 2), X.dtype),
    )(X, X, X, X, counts, after)
    return out
